# baseline (speedup 1.0000x reference)
import dataclasses
import functools

import jax
import jax.numpy as jnp
from jax import lax
from jax.experimental import pallas as pl
from jax.experimental.pallas import tpu as pltpu
from jax.experimental.pallas import tpu_sc as plsc

N = 10000
E = 320000
D_IN = 128
D_HID = 128
D_OUT = 64
N_CLASS = 2

NUM_SC = 2
NUM_SUBCORES = 16
NW = NUM_SC * NUM_SUBCORES
LANES = 16

CHUNK = 128
CH = 80
E_PAD = NW * CH * CHUNK
NBUF = 2
HALF = CH // 2
NP = N
TILE_STRIDE = 624
TILE_SPAN = 640


def _leaky_relu(x):
    return jnp.where(x >= 0, x, 0.01 * x)


@functools.lru_cache(maxsize=None)
def _make_sc_aggregate(d):
    mesh = plsc.VectorSubcoreMesh(core_axis_name="c", subcore_axis_name="s")
    nk = d // LANES

    cp = pltpu.CompilerParams()
    if "needs_layout_passes" in pltpu.CompilerParams.__dataclass_fields__:
        cp = dataclasses.replace(cp, needs_layout_passes=False)

    @functools.partial(
        pl.kernel,
        mesh=mesh,
        compiler_params=cp,
        out_type=jax.ShapeDtypeStruct((NUM_SC, NP, d), jnp.float32),
        scratch_types=(
            [
                pltpu.VMEM((HALF, CHUNK), jnp.int32),
                pltpu.VMEM((HALF, CHUNK), jnp.int32),
                pltpu.VMEM((HALF, CHUNK), jnp.float32),
            ]
            + [pltpu.VMEM((CHUNK, d), jnp.float32) for _ in range(NBUF)]
            + [pltpu.VMEM_SHARED((NP, d), jnp.float32)]
            + [pltpu.SemaphoreType.DMA for _ in range(NBUF)]
        ),
    )
    def agg(h_hbm, src_hbm, dst_hbm, w_hbm, out_hbm,
            src_v, dst_v, w_v, *rest):
        bufs = rest[:NBUF]
        acc_sh = rest[NBUF]
        gsems = rest[NBUF + 1:]
        c = lax.axis_index("c")
        s = lax.axis_index("s")
        wid = c * NUM_SUBCORES + s

        def issue_gather(jj, b):
            pltpu.async_copy(h_hbm.at[src_v.at[jj]], bufs[b], gsems[b])

        def wait_gather(jj, b):
            pltpu.make_async_copy(
                h_hbm.at[src_v.at[jj]], bufs[b], gsems[b]).wait()

        @pl.loop(0, CHUNK)
        def _(i):
            for k in range(nk):
                bufs[0][i, pl.ds(k * LANES, LANES)] = jnp.zeros(
                    (LANES,), jnp.float32)

        base = s * TILE_STRIDE
        for r in range(TILE_SPAN // CHUNK):
            pltpu.sync_copy(bufs[0], acc_sh.at[pl.ds(base + r * CHUNK,
                                                     CHUNK)])
        plsc.subcore_barrier()

        for half in range(2):
            off = half * HALF
            pltpu.sync_copy(src_hbm.at[wid, pl.ds(off, HALF)], src_v)
            pltpu.sync_copy(dst_hbm.at[wid, pl.ds(off, HALF)], dst_v)
            pltpu.sync_copy(w_hbm.at[wid, pl.ds(off, HALF)], w_v)

            @pl.loop(0, HALF)
            def _(jj):
                pltpu.async_copy(
                    h_hbm.at[src_v.at[jj]], bufs[0], gsems[0]).wait()

                @pl.loop(0, CHUNK)
                def _(i):
                    idx16 = jnp.full((LANES,), i, jnp.int32)
                    wsplat = plsc.load_gather(w_v.at[jj], [idx16])
                    for k in range(nk):
                        sl = pl.ds(k * LANES, LANES)
                        bufs[0][i, sl] = bufs[0][i, sl] * wsplat

                pltpu.sync_copy(
                    bufs[0], acc_sh.at[dst_v.at[jj]], add=True)

        plsc.subcore_barrier()

        pltpu.sync_copy(
            acc_sh.at[pl.ds(base, TILE_SPAN)],
            out_hbm.at[c, pl.ds(base, TILE_SPAN)])

    return agg


def _sc_aggregate(h, src3, dst3, w3):
    return _make_sc_aggregate(h.shape[1])(h, src3, dst3, w3)


ROW_BLK = 1000


def _dense1_body(x_ref, g_ref, b_ref, w_ref, bias_ref, o_ref):
    xv = x_ref[...]
    m = jnp.mean(xv, axis=1, keepdims=True)
    v = jnp.mean((xv - m) ** 2, axis=1, keepdims=True)
    xn = (xv - m) * lax.rsqrt(v + 1e-5) * g_ref[...] + b_ref[...]
    o_ref[...] = jnp.dot(xn, w_ref[...],
                         preferred_element_type=jnp.float32) + bias_ref[...]


def _dense1(x, ln1_g, ln1_b, fc1_W, fc1_b):
    grid = N // ROW_BLK
    return pl.pallas_call(
        _dense1_body,
        grid=(grid,),
        in_specs=[
            pl.BlockSpec((ROW_BLK, D_IN), lambda i: (i, 0)),
            pl.BlockSpec((1, D_IN), lambda i: (0, 0)),
            pl.BlockSpec((1, D_IN), lambda i: (0, 0)),
            pl.BlockSpec((D_IN, D_HID), lambda i: (0, 0)),
            pl.BlockSpec((1, D_HID), lambda i: (0, 0)),
        ],
        out_specs=pl.BlockSpec((ROW_BLK, D_HID), lambda i: (i, 0)),
        out_shape=jax.ShapeDtypeStruct((N, D_HID), jnp.float32),
    )(x, ln1_g.reshape(1, -1), ln1_b.reshape(1, -1), fc1_W,
      fc1_b.reshape(1, -1))


def _dense2_body(p_ref, h0_ref, wrel_ref, brel_ref, wroot_ref,
                 w2_ref, hp_ref):
    aggr = p_ref[0] + p_ref[1]
    h1 = (jnp.dot(aggr, wrel_ref[...], preferred_element_type=jnp.float32)
          + brel_ref[...]
          + jnp.dot(h0_ref[...], wroot_ref[...],
                    preferred_element_type=jnp.float32))
    h1 = _leaky_relu(h1)
    hp_ref[...] = jnp.dot(h1, w2_ref[...],
                          preferred_element_type=jnp.float32)


def _dense2(p, h0, W_rel1, b_rel1, W_root1, W_rel2, W_root2):
    grid = N // ROW_BLK
    w2 = jnp.concatenate([W_rel2, W_root2], axis=1)
    return pl.pallas_call(
        _dense2_body,
        grid=(grid,),
        in_specs=[
            pl.BlockSpec((2, ROW_BLK, D_HID), lambda i: (0, i, 0)),
            pl.BlockSpec((ROW_BLK, D_HID), lambda i: (i, 0)),
            pl.BlockSpec((D_HID, D_HID), lambda i: (0, 0)),
            pl.BlockSpec((1, D_HID), lambda i: (0, 0)),
            pl.BlockSpec((D_HID, D_HID), lambda i: (0, 0)),
            pl.BlockSpec((D_HID, 2 * D_OUT), lambda i: (0, 0)),
        ],
        out_specs=pl.BlockSpec((ROW_BLK, 2 * D_OUT), lambda i: (i, 0)),
        out_shape=jax.ShapeDtypeStruct((N, 2 * D_OUT), jnp.float32),
    )(p, h0, W_rel1, b_rel1.reshape(1, -1), W_root1, w2)


def _dense3_body(q_ref, hp_ref, brel2_ref, g_ref, b_ref, w_ref,
                 bias_ref, o_ref):
    h2 = (q_ref[0, :, :D_OUT] + q_ref[1, :, :D_OUT] + brel2_ref[...]
          + hp_ref[:, D_OUT:])
    h2 = _leaky_relu(h2)
    m = jnp.mean(h2, axis=1, keepdims=True)
    v = jnp.mean((h2 - m) ** 2, axis=1, keepdims=True)
    xn = (h2 - m) * lax.rsqrt(v + 1e-5) * g_ref[...] + b_ref[...]
    o_ref[...] = jnp.dot(xn, w_ref[...],
                         preferred_element_type=jnp.float32) + bias_ref[...]


def _dense3(q, hp, b_rel2, ln2_g, ln2_b, fc2_W, fc2_b):
    grid = N // ROW_BLK
    return pl.pallas_call(
        _dense3_body,
        grid=(grid,),
        in_specs=[
            pl.BlockSpec((2, ROW_BLK, 2 * D_OUT), lambda i: (0, i, 0)),
            pl.BlockSpec((ROW_BLK, 2 * D_OUT), lambda i: (i, 0)),
            pl.BlockSpec((1, D_OUT), lambda i: (0, 0)),
            pl.BlockSpec((1, D_OUT), lambda i: (0, 0)),
            pl.BlockSpec((1, D_OUT), lambda i: (0, 0)),
            pl.BlockSpec((D_OUT, N_CLASS), lambda i: (0, 0)),
            pl.BlockSpec((1, N_CLASS), lambda i: (0, 0)),
        ],
        out_specs=pl.BlockSpec((ROW_BLK, N_CLASS), lambda i: (i, 0)),
        out_shape=jax.ShapeDtypeStruct((N, N_CLASS), jnp.float32),
    )(q, hp, b_rel2.reshape(1, -1), ln2_g.reshape(1, -1),
      ln2_b.reshape(1, -1), fc2_W, fc2_b.reshape(1, -1))


def kernel(x, edge_index, edge_attr, ln1_g, ln1_b, fc1_W, fc1_b,
           W_rel1, b_rel1, W_root1, W_rel2, b_rel2, W_root2,
           ln2_g, ln2_b, fc2_W, fc2_b):
    src = edge_index[0].astype(jnp.int32)
    dst = edge_index[1].astype(jnp.int32)
    w = edge_attr[:, 0]

    pad = E_PAD - E
    src_p = jnp.concatenate([src, jnp.zeros((pad,), jnp.int32)])
    dst_p = jnp.concatenate([dst, jnp.zeros((pad,), jnp.int32)])
    w_p = jnp.concatenate([w, jnp.zeros((pad,), jnp.float32)])
    src3 = src_p.reshape(NW, CH, CHUNK)
    dst3 = dst_p.reshape(NW, CH, CHUNK)
    w3 = w_p.reshape(NW, CH, CHUNK)

    h0 = _dense1(x, ln1_g, ln1_b, fc1_W, fc1_b)
    p1 = _sc_aggregate(h0, src3, dst3, w3)
    hp = _dense2(p1, h0, W_rel1, b_rel1, W_root1, W_rel2, W_root2)
    q2 = _sc_aggregate(hp, src3, dst3, w3)
    out = _dense3(q2, hp, b_rel2, ln2_g, ln2_b, fc2_W, fc2_b)
    return out


# speedup vs baseline: 1.2822x; 1.2822x over previous
import dataclasses
import functools

import jax
import jax.numpy as jnp
from jax import lax
from jax.experimental import pallas as pl
from jax.experimental.pallas import tpu as pltpu
from jax.experimental.pallas import tpu_sc as plsc

N = 10000
E = 320000
D_IN = 128
D_HID = 128
D_OUT = 64
N_CLASS = 2

NUM_SC = 2
NUM_SUBCORES = 16
NW = NUM_SC * NUM_SUBCORES
LANES = 16

CHUNK = 128
CH = 80
E_PAD = NW * CH * CHUNK
NBUF = 2
HALF = CH // 2
NP = N
TILE_STRIDE = 624
TILE_SPAN = 640


def _leaky_relu(x):
    return jnp.where(x >= 0, x, 0.01 * x)


@functools.lru_cache(maxsize=None)
def _make_sc_aggregate(d):
    mesh = plsc.VectorSubcoreMesh(core_axis_name="c", subcore_axis_name="s")
    nk = d // LANES

    cp = pltpu.CompilerParams()
    if "needs_layout_passes" in pltpu.CompilerParams.__dataclass_fields__:
        cp = dataclasses.replace(cp, needs_layout_passes=False)

    @functools.partial(
        pl.kernel,
        mesh=mesh,
        compiler_params=cp,
        out_type=jax.ShapeDtypeStruct((NUM_SC, NP, d), jnp.float32),
        scratch_types=(
            [
                pltpu.VMEM((HALF, CHUNK), jnp.int32),
                pltpu.VMEM((HALF, CHUNK), jnp.int32),
                pltpu.VMEM((HALF, CHUNK), jnp.float32),
            ]
            + [pltpu.VMEM((CHUNK, d), jnp.float32) for _ in range(NBUF)]
            + [pltpu.VMEM_SHARED((NP, d), jnp.float32)]
            + [pltpu.SemaphoreType.DMA for _ in range(NBUF)]
        ),
    )
    def agg(h_hbm, src_hbm, dst_hbm, w_hbm, out_hbm,
            src_v, dst_v, w_v, *rest):
        bufs = rest[:NBUF]
        acc_sh = rest[NBUF]
        gsems = rest[NBUF + 1:]
        c = lax.axis_index("c")
        s = lax.axis_index("s")
        wid = c * NUM_SUBCORES + s

        def issue_gather(jj, b):
            pltpu.async_copy(h_hbm.at[src_v.at[jj]], bufs[b], gsems[b])

        def wait_gather(jj, b):
            pltpu.make_async_copy(
                h_hbm.at[src_v.at[jj]], bufs[b], gsems[b]).wait()

        @pl.loop(0, CHUNK)
        def _(i):
            for k in range(nk):
                bufs[0][i, pl.ds(k * LANES, LANES)] = jnp.zeros(
                    (LANES,), jnp.float32)

        base = s * TILE_STRIDE
        for r in range(TILE_SPAN // CHUNK):
            pltpu.sync_copy(bufs[0], acc_sh.at[pl.ds(base + r * CHUNK,
                                                     CHUNK)])
        plsc.subcore_barrier()

        def chunk_body(jj, u, refill):
            b = u % NBUF
            bn = (b + 1) % NBUF

            if refill:
                issue_gather(jj + 1, bn)
            wait_gather(jj, b)

            @plsc.parallel_loop(0, CHUNK, unroll=8)
            def _(i):
                idx16 = jnp.full((LANES,), i, jnp.int32)
                wsplat = plsc.load_gather(w_v.at[jj], [idx16])
                for k in range(nk):
                    sl = pl.ds(k * LANES, LANES)
                    bufs[b][i, sl] = bufs[b][i, sl] * wsplat

            pltpu.sync_copy(
                bufs[b], acc_sh.at[dst_v.at[jj]], add=True)

        for half in range(2):
            off = half * HALF
            pltpu.sync_copy(src_hbm.at[wid, pl.ds(off, HALF)], src_v)
            pltpu.sync_copy(dst_hbm.at[wid, pl.ds(off, HALF)], dst_v)
            pltpu.sync_copy(w_hbm.at[wid, pl.ds(off, HALF)], w_v)

            issue_gather(0, 0)
            chunk_body(0, 0, True)
            chunk_body(1, 1, True)

            @pl.loop(2, HALF - 2, step=NBUF)
            def _(j):
                for u in range(NBUF):
                    chunk_body(j + u, u, True)

            chunk_body(HALF - 2, 0, True)
            chunk_body(HALF - 1, 1, False)

        plsc.subcore_barrier()

        pltpu.sync_copy(
            acc_sh.at[pl.ds(base, TILE_SPAN)],
            out_hbm.at[c, pl.ds(base, TILE_SPAN)])

    return agg


def _sc_aggregate(h, src3, dst3, w3):
    return _make_sc_aggregate(h.shape[1])(h, src3, dst3, w3)


ROW_BLK = 1000


def _dense1_body(x_ref, g_ref, b_ref, w_ref, bias_ref, o_ref):
    xv = x_ref[...]
    m = jnp.mean(xv, axis=1, keepdims=True)
    v = jnp.mean((xv - m) ** 2, axis=1, keepdims=True)
    xn = (xv - m) * lax.rsqrt(v + 1e-5) * g_ref[...] + b_ref[...]
    o_ref[...] = jnp.dot(xn, w_ref[...],
                         preferred_element_type=jnp.float32) + bias_ref[...]


def _dense1(x, ln1_g, ln1_b, fc1_W, fc1_b):
    grid = N // ROW_BLK
    return pl.pallas_call(
        _dense1_body,
        grid=(grid,),
        in_specs=[
            pl.BlockSpec((ROW_BLK, D_IN), lambda i: (i, 0)),
            pl.BlockSpec((1, D_IN), lambda i: (0, 0)),
            pl.BlockSpec((1, D_IN), lambda i: (0, 0)),
            pl.BlockSpec((D_IN, D_HID), lambda i: (0, 0)),
            pl.BlockSpec((1, D_HID), lambda i: (0, 0)),
        ],
        out_specs=pl.BlockSpec((ROW_BLK, D_HID), lambda i: (i, 0)),
        out_shape=jax.ShapeDtypeStruct((N, D_HID), jnp.float32),
    )(x, ln1_g.reshape(1, -1), ln1_b.reshape(1, -1), fc1_W,
      fc1_b.reshape(1, -1))


def _dense2_body(p_ref, h0_ref, wrel_ref, brel_ref, wroot_ref,
                 w2_ref, hp_ref):
    aggr = p_ref[0] + p_ref[1]
    h1 = (jnp.dot(aggr, wrel_ref[...], preferred_element_type=jnp.float32)
          + brel_ref[...]
          + jnp.dot(h0_ref[...], wroot_ref[...],
                    preferred_element_type=jnp.float32))
    h1 = _leaky_relu(h1)
    hp_ref[...] = jnp.dot(h1, w2_ref[...],
                          preferred_element_type=jnp.float32)


def _dense2(p, h0, W_rel1, b_rel1, W_root1, W_rel2, W_root2):
    grid = N // ROW_BLK
    w2 = jnp.concatenate([W_rel2, W_root2], axis=1)
    return pl.pallas_call(
        _dense2_body,
        grid=(grid,),
        in_specs=[
            pl.BlockSpec((2, ROW_BLK, D_HID), lambda i: (0, i, 0)),
            pl.BlockSpec((ROW_BLK, D_HID), lambda i: (i, 0)),
            pl.BlockSpec((D_HID, D_HID), lambda i: (0, 0)),
            pl.BlockSpec((1, D_HID), lambda i: (0, 0)),
            pl.BlockSpec((D_HID, D_HID), lambda i: (0, 0)),
            pl.BlockSpec((D_HID, 2 * D_OUT), lambda i: (0, 0)),
        ],
        out_specs=pl.BlockSpec((ROW_BLK, 2 * D_OUT), lambda i: (i, 0)),
        out_shape=jax.ShapeDtypeStruct((N, 2 * D_OUT), jnp.float32),
    )(p, h0, W_rel1, b_rel1.reshape(1, -1), W_root1, w2)


def _dense3_body(q_ref, hp_ref, brel2_ref, g_ref, b_ref, w_ref,
                 bias_ref, o_ref):
    h2 = (q_ref[0, :, :D_OUT] + q_ref[1, :, :D_OUT] + brel2_ref[...]
          + hp_ref[:, D_OUT:])
    h2 = _leaky_relu(h2)
    m = jnp.mean(h2, axis=1, keepdims=True)
    v = jnp.mean((h2 - m) ** 2, axis=1, keepdims=True)
    xn = (h2 - m) * lax.rsqrt(v + 1e-5) * g_ref[...] + b_ref[...]
    o_ref[...] = jnp.dot(xn, w_ref[...],
                         preferred_element_type=jnp.float32) + bias_ref[...]


def _dense3(q, hp, b_rel2, ln2_g, ln2_b, fc2_W, fc2_b):
    grid = N // ROW_BLK
    return pl.pallas_call(
        _dense3_body,
        grid=(grid,),
        in_specs=[
            pl.BlockSpec((2, ROW_BLK, 2 * D_OUT), lambda i: (0, i, 0)),
            pl.BlockSpec((ROW_BLK, 2 * D_OUT), lambda i: (i, 0)),
            pl.BlockSpec((1, D_OUT), lambda i: (0, 0)),
            pl.BlockSpec((1, D_OUT), lambda i: (0, 0)),
            pl.BlockSpec((1, D_OUT), lambda i: (0, 0)),
            pl.BlockSpec((D_OUT, N_CLASS), lambda i: (0, 0)),
            pl.BlockSpec((1, N_CLASS), lambda i: (0, 0)),
        ],
        out_specs=pl.BlockSpec((ROW_BLK, N_CLASS), lambda i: (i, 0)),
        out_shape=jax.ShapeDtypeStruct((N, N_CLASS), jnp.float32),
    )(q, hp, b_rel2.reshape(1, -1), ln2_g.reshape(1, -1),
      ln2_b.reshape(1, -1), fc2_W, fc2_b.reshape(1, -1))


def kernel(x, edge_index, edge_attr, ln1_g, ln1_b, fc1_W, fc1_b,
           W_rel1, b_rel1, W_root1, W_rel2, b_rel2, W_root2,
           ln2_g, ln2_b, fc2_W, fc2_b):
    src = edge_index[0].astype(jnp.int32)
    dst = edge_index[1].astype(jnp.int32)
    w = edge_attr[:, 0]

    pad = E_PAD - E
    src_p = jnp.concatenate([src, jnp.zeros((pad,), jnp.int32)])
    dst_p = jnp.concatenate(
        [dst, (jnp.arange(pad, dtype=jnp.int32) * 8) % N])
    w_p = jnp.concatenate([w, jnp.zeros((pad,), jnp.float32)])
    src3 = src_p.reshape(NW, CH, CHUNK)
    dst3 = dst_p.reshape(NW, CH, CHUNK)
    w3 = w_p.reshape(NW, CH, CHUNK)

    h0 = _dense1(x, ln1_g, ln1_b, fc1_W, fc1_b)
    p1 = _sc_aggregate(h0, src3, dst3, w3)
    hp = _dense2(p1, h0, W_rel1, b_rel1, W_root1, W_rel2, W_root2)
    q2 = _sc_aggregate(hp, src3, dst3, w3)
    out = _dense3(q2, hp, b_rel2, ln2_g, ln2_b, fc2_W, fc2_b)
    return out


# speedup vs baseline: 1.4747x; 1.1501x over previous
import dataclasses
import functools

import jax
import jax.numpy as jnp
from jax import lax
from jax.experimental import pallas as pl
from jax.experimental.pallas import tpu as pltpu
from jax.experimental.pallas import tpu_sc as plsc

N = 10000
E = 320000
D_IN = 128
D_HID = 128
D_OUT = 64
N_CLASS = 2

NUM_SC = 2
NUM_SUBCORES = 16
NW = NUM_SC * NUM_SUBCORES
LANES = 16

CHUNK = 128
CH = 79
E_PAD = NW * CH * CHUNK
NP = 10240
ROWS_PER_TILE = NP // NUM_SUBCORES


def _leaky_relu(x):
    return jnp.where(x >= 0, x, 0.01 * x)


@functools.lru_cache(maxsize=None)
def _make_sc_aggregate(d):
    mesh = plsc.VectorSubcoreMesh(core_axis_name="c", subcore_axis_name="s")
    nk = d // LANES

    cp = pltpu.CompilerParams()
    if "needs_layout_passes" in pltpu.CompilerParams.__dataclass_fields__:
        cp = dataclasses.replace(cp, needs_layout_passes=False)

    @functools.partial(
        pl.kernel,
        mesh=mesh,
        compiler_params=cp,
        out_type=jax.ShapeDtypeStruct((NUM_SC, NP, d), jnp.float32),
        scratch_types=[
            pltpu.VMEM((CH, CHUNK), jnp.int32),
            pltpu.VMEM((CH, CHUNK), jnp.int32),
            pltpu.VMEM((CH, CHUNK), jnp.float32),
            pltpu.VMEM((CHUNK, d), jnp.float32),
            pltpu.VMEM_SHARED((NP, d), jnp.float32),
            pltpu.SemaphoreType.DMA,
        ],
    )
    def agg(h_hbm, src_hbm, dst_hbm, w_hbm, out_hbm,
            src_v, dst_v, w_v, rows_v, acc_sh, sem):
        c = lax.axis_index("c")
        s = lax.axis_index("s")
        wid = c * NUM_SUBCORES + s

        @pl.loop(0, CHUNK)
        def _(i):
            for k in range(nk):
                rows_v[i, pl.ds(k * LANES, LANES)] = jnp.zeros(
                    (LANES,), jnp.float32)

        base = s * ROWS_PER_TILE
        for r in range(ROWS_PER_TILE // CHUNK):
            pltpu.sync_copy(rows_v, acc_sh.at[pl.ds(base + r * CHUNK,
                                                    CHUNK)])
        plsc.subcore_barrier()

        pltpu.sync_copy(src_hbm.at[wid], src_v)
        pltpu.sync_copy(dst_hbm.at[wid], dst_v)
        pltpu.sync_copy(w_hbm.at[wid], w_v)

        @pl.loop(0, CH)
        def _(j):
            pltpu.async_copy(h_hbm.at[src_v.at[j]], rows_v, sem).wait()

            @pl.loop(0, CHUNK)
            def _(i):
                idx16 = jnp.full((LANES,), i, jnp.int32)
                wsplat = plsc.load_gather(w_v.at[j], [idx16])
                for k in range(nk):
                    sl = pl.ds(k * LANES, LANES)
                    rows_v[i, sl] = rows_v[i, sl] * wsplat

            pltpu.sync_copy(rows_v, acc_sh.at[dst_v.at[j]], add=True)

        plsc.subcore_barrier()

        pltpu.sync_copy(
            acc_sh.at[pl.ds(base, ROWS_PER_TILE)],
            out_hbm.at[c, pl.ds(base, ROWS_PER_TILE)])

    return agg


def _sc_aggregate(h, src3, dst3, w3):
    return _make_sc_aggregate(h.shape[1])(h, src3, dst3, w3)


ROW_BLK = 1000


def _dense1_body(x_ref, g_ref, b_ref, w_ref, bias_ref, o_ref):
    xv = x_ref[...]
    m = jnp.mean(xv, axis=1, keepdims=True)
    v = jnp.mean((xv - m) ** 2, axis=1, keepdims=True)
    xn = (xv - m) * lax.rsqrt(v + 1e-5) * g_ref[...] + b_ref[...]
    o_ref[...] = jnp.dot(xn, w_ref[...],
                         preferred_element_type=jnp.float32) + bias_ref[...]


def _dense1(x, ln1_g, ln1_b, fc1_W, fc1_b):
    grid = N // ROW_BLK
    return pl.pallas_call(
        _dense1_body,
        grid=(grid,),
        in_specs=[
            pl.BlockSpec((ROW_BLK, D_IN), lambda i: (i, 0)),
            pl.BlockSpec((1, D_IN), lambda i: (0, 0)),
            pl.BlockSpec((1, D_IN), lambda i: (0, 0)),
            pl.BlockSpec((D_IN, D_HID), lambda i: (0, 0)),
            pl.BlockSpec((1, D_HID), lambda i: (0, 0)),
        ],
        out_specs=pl.BlockSpec((ROW_BLK, D_HID), lambda i: (i, 0)),
        out_shape=jax.ShapeDtypeStruct((N, D_HID), jnp.float32),
    )(x, ln1_g.reshape(1, -1), ln1_b.reshape(1, -1), fc1_W,
      fc1_b.reshape(1, -1))


def _dense2_body(p_ref, h0_ref, wrel_ref, brel_ref, wroot_ref,
                 w2_ref, hp_ref):
    aggr = p_ref[0] + p_ref[1]
    h1 = (jnp.dot(aggr, wrel_ref[...], preferred_element_type=jnp.float32)
          + brel_ref[...]
          + jnp.dot(h0_ref[...], wroot_ref[...],
                    preferred_element_type=jnp.float32))
    h1 = _leaky_relu(h1)
    hp_ref[...] = jnp.dot(h1, w2_ref[...],
                          preferred_element_type=jnp.float32)


def _dense2(p, h0, W_rel1, b_rel1, W_root1, W_rel2, W_root2):
    grid = N // ROW_BLK
    w2 = jnp.concatenate([W_rel2, W_root2], axis=1)
    return pl.pallas_call(
        _dense2_body,
        grid=(grid,),
        in_specs=[
            pl.BlockSpec((2, ROW_BLK, D_HID), lambda i: (0, i, 0)),
            pl.BlockSpec((ROW_BLK, D_HID), lambda i: (i, 0)),
            pl.BlockSpec((D_HID, D_HID), lambda i: (0, 0)),
            pl.BlockSpec((1, D_HID), lambda i: (0, 0)),
            pl.BlockSpec((D_HID, D_HID), lambda i: (0, 0)),
            pl.BlockSpec((D_HID, 2 * D_OUT), lambda i: (0, 0)),
        ],
        out_specs=pl.BlockSpec((ROW_BLK, 2 * D_OUT), lambda i: (i, 0)),
        out_shape=jax.ShapeDtypeStruct((N, 2 * D_OUT), jnp.float32),
    )(p, h0, W_rel1, b_rel1.reshape(1, -1), W_root1, w2)


def _dense3_body(q_ref, hp_ref, brel2_ref, g_ref, b_ref, w_ref,
                 bias_ref, o_ref):
    h2 = (q_ref[0, :, :D_OUT] + q_ref[1, :, :D_OUT] + brel2_ref[...]
          + hp_ref[:, D_OUT:])
    h2 = _leaky_relu(h2)
    m = jnp.mean(h2, axis=1, keepdims=True)
    v = jnp.mean((h2 - m) ** 2, axis=1, keepdims=True)
    xn = (h2 - m) * lax.rsqrt(v + 1e-5) * g_ref[...] + b_ref[...]
    o_ref[...] = jnp.dot(xn, w_ref[...],
                         preferred_element_type=jnp.float32) + bias_ref[...]


def _dense3(q, hp, b_rel2, ln2_g, ln2_b, fc2_W, fc2_b):
    grid = N // ROW_BLK
    return pl.pallas_call(
        _dense3_body,
        grid=(grid,),
        in_specs=[
            pl.BlockSpec((2, ROW_BLK, 2 * D_OUT), lambda i: (0, i, 0)),
            pl.BlockSpec((ROW_BLK, 2 * D_OUT), lambda i: (i, 0)),
            pl.BlockSpec((1, D_OUT), lambda i: (0, 0)),
            pl.BlockSpec((1, D_OUT), lambda i: (0, 0)),
            pl.BlockSpec((1, D_OUT), lambda i: (0, 0)),
            pl.BlockSpec((D_OUT, N_CLASS), lambda i: (0, 0)),
            pl.BlockSpec((1, N_CLASS), lambda i: (0, 0)),
        ],
        out_specs=pl.BlockSpec((ROW_BLK, N_CLASS), lambda i: (i, 0)),
        out_shape=jax.ShapeDtypeStruct((N, N_CLASS), jnp.float32),
    )(q, hp, b_rel2.reshape(1, -1), ln2_g.reshape(1, -1),
      ln2_b.reshape(1, -1), fc2_W, fc2_b.reshape(1, -1))


def kernel(x, edge_index, edge_attr, ln1_g, ln1_b, fc1_W, fc1_b,
           W_rel1, b_rel1, W_root1, W_rel2, b_rel2, W_root2,
           ln2_g, ln2_b, fc2_W, fc2_b):
    src = edge_index[0].astype(jnp.int32)
    dst = edge_index[1].astype(jnp.int32)
    w = edge_attr[:, 0]

    pad = E_PAD - E
    src_p = jnp.concatenate([src, jnp.zeros((pad,), jnp.int32)])
    dst_p = jnp.concatenate(
        [dst, N + (jnp.arange(pad, dtype=jnp.int32) % (NP - N))])
    w_p = jnp.concatenate([w, jnp.zeros((pad,), jnp.float32)])
    src3 = src_p.reshape(NW, CH, CHUNK)
    dst3 = dst_p.reshape(NW, CH, CHUNK)
    w3 = w_p.reshape(NW, CH, CHUNK)

    h0 = _dense1(x, ln1_g, ln1_b, fc1_W, fc1_b)
    p1 = _sc_aggregate(h0, src3, dst3, w3)
    hp = _dense2(p1, h0, W_rel1, b_rel1, W_root1, W_rel2, W_root2)
    q2 = _sc_aggregate(hp, src3, dst3, w3)
    out = _dense3(q2, hp, b_rel2, ln2_g, ln2_b, fc2_W, fc2_b)
    return out
